# baseline (speedup 1.0000x reference)
import functools

import jax
import jax.numpy as jnp
from jax import lax
from jax.experimental import pallas as pl
from jax.experimental.pallas import tpu as pltpu
from jax.experimental.pallas import tpu_sc as plsc

N = 10000
D = 128
HEADS = 8
OUT = 16
TW = D + 16
NEG = -1e30

NC = 2
NS = 16
NTILES = NC * NS
B = 80

RB = 1000
NG = N // RB
ZR = 632


@functools.partial(jax.jit, static_argnames=())
def _edge_pass(tsrc, tdst, src, dst, mvec):
    E = src.shape[0]
    ept = E // NTILES
    nch = ept // B
    mesh = plsc.VectorSubcoreMesh(
        core_axis_name="c", subcore_axis_name="s", num_cores=NC, num_subcores=NS
    )

    @functools.partial(
        pl.kernel,
        out_type=jax.ShapeDtypeStruct((NC, N, TW), jnp.float32),
        mesh=mesh,
        scratch_types=[
            pltpu.VMEM_SHARED((N, TW), jnp.float32),
            pltpu.VMEM((B,), jnp.int32),
            pltpu.VMEM((B,), jnp.int32),
            pltpu.VMEM((B,), jnp.int32),
            pltpu.VMEM((B,), jnp.int32),
            pltpu.VMEM((B, TW), jnp.float32),
            pltpu.VMEM((B, TW), jnp.float32),
            pltpu.VMEM((B, 16), jnp.float32),
            pltpu.VMEM((B, 16), jnp.float32),
            pltpu.VMEM((B, TW), jnp.float32),
            pltpu.VMEM((16,), jnp.float32),
            pltpu.VMEM((16,), jnp.float32),
            pltpu.VMEM((16,), jnp.float32),
            pltpu.VMEM((16,), jnp.float32),
            pltpu.VMEM((16,), jnp.float32),
            pltpu.SemaphoreType.DMA,
            pltpu.SemaphoreType.DMA,
        ],
        compiler_params=pltpu.CompilerParams(
            needs_layout_passes=False, use_tc_tiling_on_sc=False
        ),
    )
    def edge_kernel(tsrc_hbm, tdst_hbm, src_hbm, dst_hbm, m_hbm, out_hbm,
                    acc_sh, srcv0, srcv1, dstv0, dstv1, av0, av1, dv0, dv1,
                    rv0, mv, wv, wv1, wv2, wv3, gsem0, gsem1):
        c = lax.axis_index("c")
        s = lax.axis_index("s")
        tile = c * NS + s

        def zrow(i, _):
            for j in range(TW // 16):
                rv0[i, pl.ds(j * 16, 16)] = jnp.zeros((16,), jnp.float32)
            return 0
        lax.fori_loop(0, B, zrow, 0)

        r0 = s * ZR
        for kk in range(640 // B):
            o = r0 + kk * B

            @pl.when(o + B <= N)
            def _():
                pltpu.sync_copy(rv0, acc_sh.at[pl.ds(o, B)])

        @pl.when(s == NS - 1)
        def _():
            pltpu.sync_copy(rv0.at[pl.ds(0, 40)], acc_sh.at[pl.ds(N - 40, 40)])

        def prefetch(k, srcv, dstv, av, dv, gsem):
            base = tile * ept + k * B
            pltpu.sync_copy(src_hbm.at[pl.ds(base, B)], srcv)
            pltpu.sync_copy(dst_hbm.at[pl.ds(base, B)], dstv)
            pltpu.async_copy(tsrc_hbm.at[srcv], av, gsem)
            pltpu.async_copy(tdst_hbm.at[dstv], dv, gsem)

        def wait_gathers(srcv, dstv, av, dv, gsem):
            pltpu.make_async_copy(tsrc_hbm.at[srcv], av, gsem).wait()
            pltpu.make_async_copy(tdst_hbm.at[dstv], dv, gsem).wait()

        prefetch(0, srcv0, dstv0, av0, dv0, gsem0)

        pltpu.sync_copy(m_hbm, mv)
        plsc.subcore_barrier()

        mreg = mv[...]

        def compute(av, dv, rv):
            wvs = [wv, wv1, wv2, wv3]

            def edge4(t, _):
                es = [4 * t + j for j in range(4)]
                for e, wr in zip(es, wvs):
                    ev = av[e, pl.ds(D, 16)] + dv[e, :]
                    ev = jnp.maximum(ev, ev * 0.2)
                    w = jnp.exp(ev - mreg)
                    wr[...] = w
                    rv[e, pl.ds(D, 16)] = w
                for h in range(HEADS):
                    idx = jnp.full((16,), 8 + h, jnp.int32)
                    o = pl.ds(h * OUT, OUT)
                    for e, wr in zip(es, wvs):
                        rv[e, o] = av[e, o] * plsc.load_gather(wr, [idx])
                return 0
            lax.fori_loop(0, B // 4, edge4, 0)


        def pair(i, _):
            k0 = 2 * i
            wait_gathers(srcv0, dstv0, av0, dv0, gsem0)
            prefetch(k0 + 1, srcv1, dstv1, av1, dv1, gsem1)
            compute(av0, dv0, rv0)
            pltpu.sync_copy(rv0, acc_sh.at[dstv0], add=True)

            wait_gathers(srcv1, dstv1, av1, dv1, gsem1)
            prefetch(k0 + 2, srcv0, dstv0, av0, dv0, gsem0)
            compute(av1, dv1, rv0)
            pltpu.sync_copy(rv0, acc_sh.at[dstv1], add=True)
            return 0
        lax.fori_loop(0, (nch - 1) // 2, pair, 0)

        wait_gathers(srcv0, dstv0, av0, dv0, gsem0)
        compute(av0, dv0, rv0)
        pltpu.sync_copy(rv0, acc_sh.at[dstv0], add=True)

        plsc.subcore_barrier()
        for kk in range(8):
            o = r0 + kk * B

            @pl.when(o + B <= N)
            def _():
                pltpu.sync_copy(acc_sh.at[pl.ds(o, B)],
                                out_hbm.at[c, pl.ds(o, B)])

        @pl.when(s == NS - 1)
        def _():
            pltpu.sync_copy(acc_sh.at[pl.ds(N - 40, 40)],
                            out_hbm.at[c, pl.ds(N - 40, 40)])

    return edge_kernel(tsrc, tdst, src, dst, mvec)


def _attn_tables(hh, asrc, adst, k1, tsrc_ref, tdst_ref, m_ref, i):
    rb = hh.shape[0]
    as8 = jnp.dot(hh * asrc, k1, preferred_element_type=jnp.float32)
    ad8 = jnp.dot(hh * adst, k1, preferred_element_type=jnp.float32)
    pad = jnp.full((rb, 8), NEG, jnp.float32)
    tsrc_ref[...] = jnp.concatenate([hh, pad, as8], axis=1)
    tdst_ref[...] = jnp.concatenate([pad, ad8], axis=1)

    @pl.when(i == 0)
    def _():
        m_ref[...] = jnp.full((1, 16), NEG, jnp.float32)
    m_ref[:, 0:8] = jnp.maximum(m_ref[:, 0:8],
                                jnp.max(as8, axis=0, keepdims=True))
    m_ref[:, 8:16] = jnp.maximum(m_ref[:, 8:16],
                                 jnp.max(ad8, axis=0, keepdims=True))

    @pl.when(i == NG - 1)
    def _():
        t = m_ref[:, 0:8] + m_ref[:, 8:16]
        m_ref[:, 8:16] = jnp.maximum(t, t * 0.2)
        m_ref[:, 0:8] = jnp.zeros((1, 8), jnp.float32)


def _pro_body(x_ref, wp_ref, bp_ref, w0_ref, asrc_ref, adst_ref, k1_ref,
              h_ref, tsrc_ref, tdst_ref, m_ref):
    i = pl.program_id(0)
    h = jnp.dot(x_ref[...], wp_ref[...], preferred_element_type=jnp.float32)
    h = h + bp_ref[...]
    h_ref[...] = h
    hh = jnp.dot(h, w0_ref[...], preferred_element_type=jnp.float32)
    _attn_tables(hh, asrc_ref[...], adst_ref[...], k1_ref[...],
                 tsrc_ref, tdst_ref, m_ref, i)


def _norm_block(acc_ref, res_ref, bias_ref, gamma_ref, beta_ref, k8_ref):
    a = acc_ref[0] + acc_ref[1]
    msg = a[:, 0:D]
    den = a[:, D + 8:D + 16]
    denb = jnp.dot(den, k8_ref[...], preferred_element_type=jnp.float32)
    o = msg / (denb + 1e-16)
    h2 = jnp.maximum(o + bias_ref[...], 0.0)
    u = h2 + res_ref[...]
    mu = jnp.mean(u, axis=1, keepdims=True)
    d = u - mu
    var = jnp.mean(d * d, axis=1, keepdims=True)
    return d * lax.rsqrt(var + 1e-5) * gamma_ref[...] + beta_ref[...]


def _mid_body(acc_ref, res_ref, bias_ref, gamma_ref, beta_ref, k8_ref,
              wn_ref, asrc_ref, adst_ref, k1_ref,
              h_ref, tsrc_ref, tdst_ref, m_ref):
    i = pl.program_id(0)
    hn = _norm_block(acc_ref, res_ref, bias_ref, gamma_ref, beta_ref, k8_ref)
    h_ref[...] = hn
    hh = jnp.dot(hn, wn_ref[...], preferred_element_type=jnp.float32)
    _attn_tables(hh, asrc_ref[...], adst_ref[...], k1_ref[...],
                 tsrc_ref, tdst_ref, m_ref, i)


def _epi_body(acc_ref, res_ref, bias_ref, gamma_ref, beta_ref, k8_ref, h_ref):
    h_ref[...] = _norm_block(acc_ref, res_ref, bias_ref, gamma_ref, beta_ref,
                             k8_ref)


def _row_spec(w):
    return pl.BlockSpec((RB, w), lambda i: (i, 0))


def _full_spec(shape):
    nd = len(shape)
    return pl.BlockSpec(shape, lambda i: (0,) * nd)


_OUT_SPECS_TBL = [
    _row_spec(D),
    _row_spec(TW),
    _row_spec(16),
    pl.BlockSpec((1, 16), lambda i: (0, 0)),
]

_OUT_SHAPES_TBL = [
    jax.ShapeDtypeStruct((N, D), jnp.float32),
    jax.ShapeDtypeStruct((N, TW), jnp.float32),
    jax.ShapeDtypeStruct((N, 16), jnp.float32),
    jax.ShapeDtypeStruct((1, 16), jnp.float32),
]


def _tc_pro(x, Wp, bp, W0, asrc, adst, k1):
    return pl.pallas_call(
        _pro_body,
        grid=(NG,),
        in_specs=[
            _row_spec(D),
            _full_spec((D, D)),
            _full_spec((1, D)),
            _full_spec((D, D)),
            _full_spec((1, D)),
            _full_spec((1, D)),
            _full_spec((D, HEADS)),
        ],
        out_specs=_OUT_SPECS_TBL,
        out_shape=_OUT_SHAPES_TBL,
    )(x, Wp, bp, W0, asrc, adst, k1)


def _tc_mid(acc, res, bias, gamma, beta, k8, Wn, asrc, adst, k1):
    return pl.pallas_call(
        _mid_body,
        grid=(NG,),
        in_specs=[
            pl.BlockSpec((NC, RB, TW), lambda i: (0, i, 0)),
            _row_spec(D),
            _full_spec((1, D)),
            _full_spec((1, D)),
            _full_spec((1, D)),
            _full_spec((HEADS, D)),
            _full_spec((D, D)),
            _full_spec((1, D)),
            _full_spec((1, D)),
            _full_spec((D, HEADS)),
        ],
        out_specs=_OUT_SPECS_TBL,
        out_shape=_OUT_SHAPES_TBL,
    )(acc, res, bias, gamma, beta, k8, Wn, asrc, adst, k1)


def _tc_epi(acc, res, bias, gamma, beta, k8):
    return pl.pallas_call(
        _epi_body,
        grid=(NG,),
        in_specs=[
            pl.BlockSpec((NC, RB, TW), lambda i: (0, i, 0)),
            _row_spec(D),
            _full_spec((1, D)),
            _full_spec((1, D)),
            _full_spec((1, D)),
            _full_spec((HEADS, D)),
        ],
        out_specs=_row_spec(D),
        out_shape=jax.ShapeDtypeStruct((N, D), jnp.float32),
    )(acc, res, bias, gamma, beta, k8)


def kernel(x, edge_index, node_types, Wp, bp, Ws, att_src, att_dst, biases,
           gammas, betas):
    src = edge_index[0]
    dst = edge_index[1]

    head_of = jnp.arange(D, dtype=jnp.int32) // OUT
    k8 = (head_of[None, :] == jnp.arange(HEADS, dtype=jnp.int32)[:, None])
    k8 = k8.astype(jnp.float32)
    k1 = k8.T

    h, tsrc, tdst, m = _tc_pro(
        x, Wp, bp.reshape(1, D), Ws[0],
        att_src[0].reshape(1, D), att_dst[0].reshape(1, D), k1)

    for l in range(3):
        acc = _edge_pass(tsrc, tdst, src, dst, m.reshape(16))
        if l < 2:
            h, tsrc, tdst, m = _tc_mid(
                acc, h, biases[l].reshape(1, D), gammas[l].reshape(1, D),
                betas[l].reshape(1, D), k8, Ws[l + 1],
                att_src[l + 1].reshape(1, D), att_dst[l + 1].reshape(1, D), k1)
        else:
            h = _tc_epi(acc, h, biases[l].reshape(1, D),
                        gammas[l].reshape(1, D), betas[l].reshape(1, D), k8)
    return h


# speedup vs baseline: 1.3737x; 1.3737x over previous
import functools

import jax
import jax.numpy as jnp
from jax import lax
from jax.experimental import pallas as pl
from jax.experimental.pallas import tpu as pltpu
from jax.experimental.pallas import tpu_sc as plsc

N = 10000
D = 128
HEADS = 8
OUT = 16
TW = D + 16
NEG = -1e30

NC = 2
NS = 16
NTILES = NC * NS
B = 80

RB = 1000
NG = N // RB
ZR = 632


@functools.partial(jax.jit, static_argnames=())
def _edge_pass(tsrc, tdst, src, dst, mvec):
    E = src.shape[0]
    ept = E // NTILES
    nch = ept // B
    mesh = plsc.VectorSubcoreMesh(
        core_axis_name="c", subcore_axis_name="s", num_cores=NC, num_subcores=NS
    )

    @functools.partial(
        pl.kernel,
        out_type=jax.ShapeDtypeStruct((NC, N, TW), jnp.float32),
        mesh=mesh,
        scratch_types=[
            pltpu.VMEM_SHARED((N, TW), jnp.float32),
            pltpu.VMEM((B,), jnp.int32),
            pltpu.VMEM((B,), jnp.int32),
            pltpu.VMEM((B,), jnp.int32),
            pltpu.VMEM((B,), jnp.int32),
            pltpu.VMEM((B, TW), jnp.float32),
            pltpu.VMEM((B, TW), jnp.float32),
            pltpu.VMEM((B, 16), jnp.float32),
            pltpu.VMEM((B, 16), jnp.float32),
            pltpu.VMEM((B, TW), jnp.float32),
            pltpu.VMEM((16,), jnp.float32),
            pltpu.VMEM((16,), jnp.float32),
            pltpu.VMEM((16,), jnp.float32),
            pltpu.SemaphoreType.DMA,
            pltpu.SemaphoreType.DMA,
        ],
        compiler_params=pltpu.CompilerParams(
            needs_layout_passes=False, use_tc_tiling_on_sc=False
        ),
    )
    def edge_kernel(tsrc_hbm, tdst_hbm, src_hbm, dst_hbm, m_hbm, out_hbm,
                    acc_sh, srcv0, srcv1, dstv0, dstv1, av0, av1, dv0, dv1,
                    rv0, mv, wv, wv1, gsem0, gsem1):
        c = lax.axis_index("c")
        s = lax.axis_index("s")
        tile = c * NS + s

        def zrow(i, _):
            for j in range(TW // 16):
                rv0[i, pl.ds(j * 16, 16)] = jnp.zeros((16,), jnp.float32)
            return 0
        lax.fori_loop(0, B, zrow, 0)

        r0 = s * ZR
        for kk in range(640 // B):
            o = r0 + kk * B

            @pl.when(o + B <= N)
            def _():
                pltpu.sync_copy(rv0, acc_sh.at[pl.ds(o, B)])

        @pl.when(s == NS - 1)
        def _():
            pltpu.sync_copy(rv0.at[pl.ds(0, 40)], acc_sh.at[pl.ds(N - 40, 40)])

        def prefetch(k, srcv, dstv, av, dv, gsem):
            base = tile * ept + k * B
            pltpu.sync_copy(src_hbm.at[pl.ds(base, B)], srcv)
            pltpu.sync_copy(dst_hbm.at[pl.ds(base, B)], dstv)
            pltpu.async_copy(tsrc_hbm.at[srcv], av, gsem)
            pltpu.async_copy(tdst_hbm.at[dstv], dv, gsem)

        def wait_gathers(srcv, dstv, av, dv, gsem):
            pltpu.make_async_copy(tsrc_hbm.at[srcv], av, gsem).wait()
            pltpu.make_async_copy(tdst_hbm.at[dstv], dv, gsem).wait()

        prefetch(0, srcv0, dstv0, av0, dv0, gsem0)

        pltpu.sync_copy(m_hbm, mv)
        plsc.subcore_barrier()

        mreg = mv[...]

        def compute(av, dv, rv):
            def edge2(t, _):
                e0 = 2 * t
                e1 = e0 + 1
                ev0 = av[e0, pl.ds(D, 16)] + dv[e0, :]
                ev1 = av[e1, pl.ds(D, 16)] + dv[e1, :]
                ev0 = jnp.maximum(ev0, ev0 * 0.2)
                ev1 = jnp.maximum(ev1, ev1 * 0.2)
                w0 = jnp.exp(ev0 - mreg)
                w1 = jnp.exp(ev1 - mreg)
                wv[...] = w0
                wv1[...] = w1
                rv[e0, pl.ds(D, 16)] = w0
                rv[e1, pl.ds(D, 16)] = w1
                for h in range(HEADS):
                    idx = jnp.full((16,), 8 + h, jnp.int32)
                    wb0 = plsc.load_gather(wv, [idx])
                    wb1 = plsc.load_gather(wv1, [idx])
                    o = pl.ds(h * OUT, OUT)
                    rv[e0, o] = av[e0, o] * wb0
                    rv[e1, o] = av[e1, o] * wb1
                return 0
            lax.fori_loop(0, B // 2, edge2, 0)


        def pair(i, _):
            k0 = 2 * i
            wait_gathers(srcv0, dstv0, av0, dv0, gsem0)
            prefetch(k0 + 1, srcv1, dstv1, av1, dv1, gsem1)
            compute(av0, dv0, rv0)
            pltpu.sync_copy(rv0, acc_sh.at[dstv0], add=True)

            wait_gathers(srcv1, dstv1, av1, dv1, gsem1)
            prefetch(k0 + 2, srcv0, dstv0, av0, dv0, gsem0)
            compute(av1, dv1, rv0)
            pltpu.sync_copy(rv0, acc_sh.at[dstv1], add=True)
            return 0
        lax.fori_loop(0, (nch - 1) // 2, pair, 0)

        wait_gathers(srcv0, dstv0, av0, dv0, gsem0)
        compute(av0, dv0, rv0)
        pltpu.sync_copy(rv0, acc_sh.at[dstv0], add=True)

        plsc.subcore_barrier()
        for kk in range(8):
            o = r0 + kk * B

            @pl.when(o + B <= N)
            def _():
                pltpu.sync_copy(acc_sh.at[pl.ds(o, B)],
                                out_hbm.at[c, pl.ds(o, B)])

        @pl.when(s == NS - 1)
        def _():
            pltpu.sync_copy(acc_sh.at[pl.ds(N - 40, 40)],
                            out_hbm.at[c, pl.ds(N - 40, 40)])

    return edge_kernel(tsrc, tdst, src, dst, mvec)


def _attn_tables(hh, asrc, adst, k1, tsrc_ref, tdst_ref, m_ref, i):
    rb = hh.shape[0]
    as8 = jnp.dot(hh * asrc, k1, preferred_element_type=jnp.float32)
    ad8 = jnp.dot(hh * adst, k1, preferred_element_type=jnp.float32)
    pad = jnp.full((rb, 8), NEG, jnp.float32)
    tsrc_ref[...] = jnp.concatenate([hh, pad, as8], axis=1)
    tdst_ref[...] = jnp.concatenate([pad, ad8], axis=1)

    @pl.when(i == 0)
    def _():
        m_ref[...] = jnp.full((1, 16), NEG, jnp.float32)
    m_ref[:, 0:8] = jnp.maximum(m_ref[:, 0:8],
                                jnp.max(as8, axis=0, keepdims=True))
    m_ref[:, 8:16] = jnp.maximum(m_ref[:, 8:16],
                                 jnp.max(ad8, axis=0, keepdims=True))

    @pl.when(i == NG - 1)
    def _():
        t = m_ref[:, 0:8] + m_ref[:, 8:16]
        m_ref[:, 8:16] = jnp.maximum(t, t * 0.2)
        m_ref[:, 0:8] = jnp.zeros((1, 8), jnp.float32)


def _pro_body(x_ref, wp_ref, bp_ref, w0_ref, asrc_ref, adst_ref, k1_ref,
              h_ref, tsrc_ref, tdst_ref, m_ref):
    i = pl.program_id(0)
    h = jnp.dot(x_ref[...], wp_ref[...], preferred_element_type=jnp.float32)
    h = h + bp_ref[...]
    h_ref[...] = h
    hh = jnp.dot(h, w0_ref[...], preferred_element_type=jnp.float32)
    _attn_tables(hh, asrc_ref[...], adst_ref[...], k1_ref[...],
                 tsrc_ref, tdst_ref, m_ref, i)


def _norm_block(acc_ref, res_ref, bias_ref, gamma_ref, beta_ref, k8_ref):
    a = acc_ref[0] + acc_ref[1]
    msg = a[:, 0:D]
    den = a[:, D + 8:D + 16]
    denb = jnp.dot(den, k8_ref[...], preferred_element_type=jnp.float32)
    o = msg / (denb + 1e-16)
    h2 = jnp.maximum(o + bias_ref[...], 0.0)
    u = h2 + res_ref[...]
    mu = jnp.mean(u, axis=1, keepdims=True)
    d = u - mu
    var = jnp.mean(d * d, axis=1, keepdims=True)
    return d * lax.rsqrt(var + 1e-5) * gamma_ref[...] + beta_ref[...]


def _mid_body(acc_ref, res_ref, bias_ref, gamma_ref, beta_ref, k8_ref,
              wn_ref, asrc_ref, adst_ref, k1_ref,
              h_ref, tsrc_ref, tdst_ref, m_ref):
    i = pl.program_id(0)
    hn = _norm_block(acc_ref, res_ref, bias_ref, gamma_ref, beta_ref, k8_ref)
    h_ref[...] = hn
    hh = jnp.dot(hn, wn_ref[...], preferred_element_type=jnp.float32)
    _attn_tables(hh, asrc_ref[...], adst_ref[...], k1_ref[...],
                 tsrc_ref, tdst_ref, m_ref, i)


def _epi_body(acc_ref, res_ref, bias_ref, gamma_ref, beta_ref, k8_ref, h_ref):
    h_ref[...] = _norm_block(acc_ref, res_ref, bias_ref, gamma_ref, beta_ref,
                             k8_ref)


def _row_spec(w):
    return pl.BlockSpec((RB, w), lambda i: (i, 0))


def _full_spec(shape):
    nd = len(shape)
    return pl.BlockSpec(shape, lambda i: (0,) * nd)


_OUT_SPECS_TBL = [
    _row_spec(D),
    _row_spec(TW),
    _row_spec(16),
    pl.BlockSpec((1, 16), lambda i: (0, 0)),
]

_OUT_SHAPES_TBL = [
    jax.ShapeDtypeStruct((N, D), jnp.float32),
    jax.ShapeDtypeStruct((N, TW), jnp.float32),
    jax.ShapeDtypeStruct((N, 16), jnp.float32),
    jax.ShapeDtypeStruct((1, 16), jnp.float32),
]


def _tc_pro(x, Wp, bp, W0, asrc, adst, k1):
    return pl.pallas_call(
        _pro_body,
        grid=(NG,),
        in_specs=[
            _row_spec(D),
            _full_spec((D, D)),
            _full_spec((1, D)),
            _full_spec((D, D)),
            _full_spec((1, D)),
            _full_spec((1, D)),
            _full_spec((D, HEADS)),
        ],
        out_specs=_OUT_SPECS_TBL,
        out_shape=_OUT_SHAPES_TBL,
    )(x, Wp, bp, W0, asrc, adst, k1)


def _tc_mid(acc, res, bias, gamma, beta, k8, Wn, asrc, adst, k1):
    return pl.pallas_call(
        _mid_body,
        grid=(NG,),
        in_specs=[
            pl.BlockSpec((NC, RB, TW), lambda i: (0, i, 0)),
            _row_spec(D),
            _full_spec((1, D)),
            _full_spec((1, D)),
            _full_spec((1, D)),
            _full_spec((HEADS, D)),
            _full_spec((D, D)),
            _full_spec((1, D)),
            _full_spec((1, D)),
            _full_spec((D, HEADS)),
        ],
        out_specs=_OUT_SPECS_TBL,
        out_shape=_OUT_SHAPES_TBL,
    )(acc, res, bias, gamma, beta, k8, Wn, asrc, adst, k1)


def _tc_epi(acc, res, bias, gamma, beta, k8):
    return pl.pallas_call(
        _epi_body,
        grid=(NG,),
        in_specs=[
            pl.BlockSpec((NC, RB, TW), lambda i: (0, i, 0)),
            _row_spec(D),
            _full_spec((1, D)),
            _full_spec((1, D)),
            _full_spec((1, D)),
            _full_spec((HEADS, D)),
        ],
        out_specs=_row_spec(D),
        out_shape=jax.ShapeDtypeStruct((N, D), jnp.float32),
    )(acc, res, bias, gamma, beta, k8)


def kernel(x, edge_index, node_types, Wp, bp, Ws, att_src, att_dst, biases,
           gammas, betas):
    src = edge_index[0]
    dst = edge_index[1]

    head_of = jnp.arange(D, dtype=jnp.int32) // OUT
    k8 = (head_of[None, :] == jnp.arange(HEADS, dtype=jnp.int32)[:, None])
    k8 = k8.astype(jnp.float32)
    k1 = k8.T

    h, tsrc, tdst, m = _tc_pro(
        x, Wp, bp.reshape(1, D), Ws[0],
        att_src[0].reshape(1, D), att_dst[0].reshape(1, D), k1)

    for l in range(3):
        acc = _edge_pass(tsrc, tdst, src, dst, m.reshape(16))
        if l < 2:
            h, tsrc, tdst, m = _tc_mid(
                acc, h, biases[l].reshape(1, D), gammas[l].reshape(1, D),
                betas[l].reshape(1, D), k8, Ws[l + 1],
                att_src[l + 1].reshape(1, D), att_dst[l + 1].reshape(1, D), k1)
        else:
            h = _tc_epi(acc, h, biases[l].reshape(1, D),
                        gammas[l].reshape(1, D), betas[l].reshape(1, D), k8)
    return h
